# baseline (speedup 1.0000x reference)
import functools

import jax
import jax.numpy as jnp
from jax import lax
from jax.experimental import pallas as pl
from jax.experimental.pallas import tpu as pltpu
from jax.experimental.pallas import tpu_sc as plsc

HIDDEN = 64
NC = 2
NS = 16
NW = NC * NS
CHUNK = 128


def kernel(input_ids, token_embedding_weight):
    B, S = input_ids.shape
    total = B * S
    per_w = total // NW
    n_chunks = per_w // CHUNK
    idx = input_ids.reshape(NW, n_chunks, CHUNK).astype(jnp.int32)

    mesh = plsc.VectorSubcoreMesh(core_axis_name="c", subcore_axis_name="s")

    @functools.partial(
        pl.kernel,
        mesh=mesh,
        out_type=jax.ShapeDtypeStruct((total, HIDDEN), jnp.float32),
        compiler_params=pltpu.CompilerParams(use_tc_tiling_on_sc=False),
        scratch_types=[
            pltpu.VMEM((n_chunks, CHUNK), jnp.int32),
            pltpu.VMEM((2, CHUNK, HIDDEN), jnp.float32),
            pltpu.SemaphoreType.DMA,
            pltpu.SemaphoreType.DMA,
        ],
    )
    def emb(idx_hbm, table_hbm, out_hbm, idx_v, rows_v, gsem, osem):
        wid = lax.axis_index("s") * NC + lax.axis_index("c")
        base = wid * per_w
        pltpu.sync_copy(idx_hbm.at[wid], idx_v)

        pltpu.async_copy(table_hbm.at[idx_v.at[0]], rows_v.at[0], gsem)

        def body(g, _):
            j0 = g * 2
            for b in range(2):
                j = j0 + b
                pltpu.make_async_copy(
                    table_hbm.at[idx_v.at[j]], rows_v.at[b], gsem
                ).wait()

                @pl.when(j + 1 < n_chunks)
                def _():
                    pltpu.async_copy(
                        table_hbm.at[idx_v.at[j + 1]], rows_v.at[1 - b], gsem
                    )

                pltpu.async_copy(
                    rows_v.at[b],
                    out_hbm.at[pl.ds(base + j * CHUNK, CHUNK)],
                    osem,
                ).wait()
            return 0

        lax.fori_loop(0, n_chunks // 2, body, 0)

    out = emb(idx, token_embedding_weight)
    return out.reshape(B, S, HIDDEN)


# speedup vs baseline: 1.0713x; 1.0713x over previous
import functools

import jax
import jax.numpy as jnp
from jax import lax
from jax.experimental import pallas as pl
from jax.experimental.pallas import tpu as pltpu
from jax.experimental.pallas import tpu_sc as plsc

HIDDEN = 64
NC = 2
NS = 16
NW = NC * NS
CHUNK = 128


def kernel(input_ids, token_embedding_weight):
    B, S = input_ids.shape
    total = B * S
    per_w = total // NW
    n_chunks = per_w // CHUNK
    idx = input_ids.reshape(NW, n_chunks, CHUNK).astype(jnp.int32)

    mesh = plsc.VectorSubcoreMesh(core_axis_name="c", subcore_axis_name="s")

    GPC = 4
    group = GPC * CHUNK
    n_groups = per_w // group

    @functools.partial(
        pl.kernel,
        mesh=mesh,
        out_type=jax.ShapeDtypeStruct((total, HIDDEN), jnp.float32),
        compiler_params=pltpu.CompilerParams(use_tc_tiling_on_sc=False),
        scratch_types=[
            pltpu.VMEM((n_chunks, CHUNK), jnp.int32),
            pltpu.VMEM((2, group, HIDDEN), jnp.float32),
            pltpu.SemaphoreType.DMA,
            pltpu.SemaphoreType.DMA,
        ],
    )
    def emb(idx_hbm, table_hbm, out_hbm, idx_v, rows_v, gsem, osem):
        wid = lax.axis_index("s") * NC + lax.axis_index("c")
        base = wid * per_w
        pltpu.sync_copy(idx_hbm.at[wid], idx_v)

        def fire(g, p):
            for b in range(GPC):
                pltpu.async_copy(
                    table_hbm.at[idx_v.at[g * GPC + b]],
                    rows_v.at[p, pl.ds(b * CHUNK, CHUNK)],
                    gsem,
                )

        fire(0, 0)

        def body(g, _):
            p = lax.rem(g, 2)
            for b in range(GPC):
                pltpu.make_async_copy(
                    table_hbm.at[idx_v.at[g * GPC + b]],
                    rows_v.at[p, pl.ds(b * CHUNK, CHUNK)],
                    gsem,
                ).wait()

            @pl.when(g + 1 < n_groups)
            def _():
                fire(g + 1, 1 - p)

            pltpu.async_copy(
                rows_v.at[p],
                out_hbm.at[pl.ds(base + g * group, group)],
                osem,
            ).wait()
            return 0

        lax.fori_loop(0, n_groups, body, 0)

    out = emb(idx, token_embedding_weight)
    return out.reshape(B, S, HIDDEN)


# speedup vs baseline: 1.3074x; 1.2205x over previous
import functools

import jax
import jax.numpy as jnp
from jax import lax
from jax.experimental import pallas as pl
from jax.experimental.pallas import tpu as pltpu
from jax.experimental.pallas import tpu_sc as plsc

HIDDEN = 64
PADDED = 128
NC = 2
NS = 16
NW = NC * NS
CHUNK = 128


def kernel(input_ids, token_embedding_weight):
    B, S = input_ids.shape
    V = token_embedding_weight.shape[0]
    total = B * S
    per_w = total // NW
    n_chunks = per_w // CHUNK
    idx = input_ids.reshape(NW, n_chunks, CHUNK).astype(jnp.int32)
    table128 = jnp.pad(token_embedding_weight, ((0, 0), (0, PADDED - HIDDEN)))

    mesh = plsc.VectorSubcoreMesh(core_axis_name="c", subcore_axis_name="s")

    GPC = 2
    group = GPC * CHUNK
    n_groups = per_w // group

    @functools.partial(
        pl.kernel,
        mesh=mesh,
        out_type=jax.ShapeDtypeStruct((total, PADDED), jnp.float32),
        scratch_types=[
            pltpu.VMEM((n_chunks, CHUNK), jnp.int32),
            pltpu.VMEM((2, group, PADDED), jnp.float32),
            pltpu.SemaphoreType.DMA,
            pltpu.SemaphoreType.DMA,
        ],
    )
    def emb(idx_hbm, table_hbm, out_hbm, idx_v, rows_v, gsem, osem):
        wid = lax.axis_index("s") * NC + lax.axis_index("c")
        base = wid * per_w
        pltpu.sync_copy(idx_hbm.at[wid], idx_v)

        def fire(g, p):
            for b in range(GPC):
                pltpu.async_copy(
                    table_hbm.at[idx_v.at[g * GPC + b]],
                    rows_v.at[p, pl.ds(b * CHUNK, CHUNK)],
                    gsem,
                )

        fire(0, 0)

        def body(g, _):
            p = lax.rem(g, 2)
            for b in range(GPC):
                pltpu.make_async_copy(
                    table_hbm.at[idx_v.at[g * GPC + b]],
                    rows_v.at[p, pl.ds(b * CHUNK, CHUNK)],
                    gsem,
                ).wait()

            @pl.when(g + 1 < n_groups)
            def _():
                fire(g + 1, 1 - p)

            pltpu.async_copy(
                rows_v.at[p],
                out_hbm.at[pl.ds(base + g * group, group)],
                osem,
            ).wait()
            return 0

        lax.fori_loop(0, n_groups, body, 0)

    out = emb(idx, table128)
    return out[:, :HIDDEN].reshape(B, S, HIDDEN)
